# Initial kernel scaffold; baseline (speedup 1.0000x reference)
import jax
import jax.numpy as jnp
from jax.experimental import pallas as pl


def kernel(x, W1s, W2s, W3s, W1c, W2c, W3c, Was, Wac, Wa):
    raise NotImplementedError("write your pallas kernel here")


# speedup vs baseline: 30.0941x; 30.0941x over previous
import functools

import jax
import jax.numpy as jnp
from jax import lax
from jax.experimental import pallas as pl
from jax.experimental.pallas import tpu as pltpu
from jax.experimental.pallas import tpu_sc as plsc

NPTS = 4096
KNB = 5
_NC, _NS = 2, 16
_NW = _NC * _NS
_PPW = NPTS // _NW


def _knn_body(xr_ref, xa_ref, idx_ref):
    xr = xr_ref[...]
    xa = xa_ref[...]
    d2 = lax.dot_general(xr.astype(jnp.bfloat16), xa.astype(jnp.bfloat16),
                         (((1,), (1,)), ((), ())),
                         preferred_element_type=jnp.float32)
    nr = jnp.sum(xr * xr, axis=1, keepdims=True)
    na = jnp.sum(xa * xa, axis=1)[None, :]
    p = 2.0 * d2 - nr - na
    iota = lax.broadcasted_iota(jnp.int32, p.shape, 1)
    cols = []
    for t in range(KNB):
        m = jnp.max(p, axis=1, keepdims=True)
        cand = jnp.where(p == m, iota, NPTS)
        a = jnp.min(cand, axis=1, keepdims=True)
        cols.append(a)
        if t < KNB - 1:
            p = jnp.where(iota == a, -jnp.inf, p)
    idx_ref[...] = jnp.concatenate(cols, axis=1)


def _knn(f, rblk=512):
    n, c = f.shape
    return pl.pallas_call(
        _knn_body,
        grid=(n // rblk,),
        in_specs=[pl.BlockSpec((rblk, c), lambda i: (i, 0)),
                  pl.BlockSpec((n, c), lambda i: (0, 0))],
        out_specs=pl.BlockSpec((rblk, KNB), lambda i: (i, 0)),
        out_shape=jax.ShapeDtypeStruct((n, KNB), jnp.int32),
    )(f, f)


def _gh_body(f_ref, w1_ref, w2_ref, g_ref, h_ref):
    f = f_ref[...]
    w1 = w1_ref[...]
    w2 = w2_ref[...]
    g_ref[...] = lax.dot_general(f, w1, (((1,), (1,)), ((), ())),
                                 preferred_element_type=jnp.float32)
    h_ref[...] = lax.dot_general(f, w2 - w1, (((1,), (1,)), ((), ())),
                                 preferred_element_type=jnp.float32)


def _gh(f, w1, w2):
    n = f.shape[0]
    o = w1.shape[0]
    return pl.pallas_call(
        _gh_body,
        out_shape=[jax.ShapeDtypeStruct((n, o), jnp.float32)] * 2,
    )(f, w1, w2)


@functools.cache
def _gather_stats_fn(o):
    ch = 8192 // o
    nch = _PPW // ch
    mesh = plsc.VectorSubcoreMesh(core_axis_name="c", subcore_axis_name="s")

    @functools.partial(
        pl.kernel,
        out_type=[jax.ShapeDtypeStruct((NPTS, o), jnp.float32)] * 3,
        mesh=mesh,
        compiler_params=pltpu.CompilerParams(use_tc_tiling_on_sc=False),
        scratch_types=[
            pltpu.VMEM((KNB, ch), jnp.int32),
            pltpu.VMEM((KNB, ch, o), jnp.float32),
            pltpu.VMEM((ch, o), jnp.float32),
            pltpu.VMEM((ch, o), jnp.float32),
            pltpu.VMEM((ch, o), jnp.float32),
            pltpu.SemaphoreType.DMA,
        ],
    )
    def kern(idxt_hbm, g_hbm, m_hbm, s1_hbm, s2_hbm,
             idx_v, rows, m_st, s1_st, s2_st, sem):
        wid = lax.axis_index("s") * _NC + lax.axis_index("c")
        for ci in range(nch):
            cbase = wid * _PPW + ci * ch
            for j in range(KNB):
                pltpu.sync_copy(idxt_hbm.at[pl.ds(j * NPTS + cbase, ch)],
                                idx_v.at[j])
            cps = [pltpu.async_copy(g_hbm.at[idx_v.at[j]], rows.at[j], sem)
                   for j in range(KNB)]
            for cp in cps:
                cp.wait()

            def pbody(pp, carry):
                for cc in range(o // 16):
                    sl = pl.ds(cc * 16, 16)
                    v0 = rows[0, pp, sl]
                    mv = v0
                    sv = v0
                    qv = v0 * v0
                    for j in range(1, KNB):
                        v = rows[j, pp, sl]
                        mv = jnp.maximum(mv, v)
                        sv = sv + v
                        qv = qv + v * v
                    m_st[pp, sl] = mv
                    s1_st[pp, sl] = sv
                    s2_st[pp, sl] = qv
                return carry

            lax.fori_loop(0, ch, pbody, 0)
            pltpu.sync_copy(m_st, m_hbm.at[pl.ds(cbase, ch)])
            pltpu.sync_copy(s1_st, s1_hbm.at[pl.ds(cbase, ch)])
            pltpu.sync_copy(s2_st, s2_hbm.at[pl.ds(cbase, ch)])

    return kern


@functools.cache
def _gather_rows_fn(c):
    mesh = plsc.VectorSubcoreMesh(core_axis_name="c", subcore_axis_name="s")

    @functools.partial(
        pl.kernel,
        out_type=jax.ShapeDtypeStruct((KNB * NPTS, c), jnp.float32),
        mesh=mesh,
        compiler_params=pltpu.CompilerParams(use_tc_tiling_on_sc=False),
        scratch_types=[
            pltpu.VMEM((KNB, _PPW), jnp.int32),
            pltpu.VMEM((KNB, _PPW, c), jnp.float32),
            pltpu.SemaphoreType.DMA,
        ],
    )
    def kern(idxt_hbm, f_hbm, fj_hbm, idx_v, rows, sem):
        wid = lax.axis_index("s") * _NC + lax.axis_index("c")
        base = wid * _PPW
        for j in range(KNB):
            pltpu.sync_copy(idxt_hbm.at[pl.ds(j * NPTS + base, _PPW)],
                            idx_v.at[j])
        cps = [pltpu.async_copy(f_hbm.at[idx_v.at[j]], rows.at[j], sem)
               for j in range(KNB)]
        for cp in cps:
            cp.wait()
        for j in range(KNB):
            pltpu.sync_copy(rows.at[j], fj_hbm.at[pl.ds(j * NPTS + base, _PPW)])

    return kern


def _edge_exact_stage1(fj3, f, w, c_real, rblk=512):
    n, cp = f.shape
    o = w.shape[0]
    nb = n // rblk

    def body(fj_ref, f_ref, w_ref, my_ref, ps_ref, pq_ref):
        fj = fj_ref[...][:, :, :c_real]
        fv = f_ref[...][:, :c_real]
        d = fj - fv[None]
        e = jnp.concatenate(
            [d, jnp.broadcast_to(fv[None], (KNB, rblk, c_real))], axis=2)
        e2 = e.reshape(KNB * rblk, 2 * c_real)
        y = lax.dot_general(e2.astype(jnp.bfloat16),
                            w_ref[...].astype(jnp.bfloat16),
                            (((1,), (1,)), ((), ())),
                            preferred_element_type=jnp.float32)
        my_ref[...] = jnp.max(y.reshape(KNB, rblk, o), axis=0)
        ps_ref[...] = jnp.sum(y, axis=0, keepdims=True)[None]
        pq_ref[...] = jnp.sum(y * y, axis=0, keepdims=True)[None]

    return pl.pallas_call(
        body,
        grid=(nb,),
        in_specs=[pl.BlockSpec((KNB, rblk, cp), lambda i: (0, i, 0)),
                  pl.BlockSpec((rblk, cp), lambda i: (i, 0)),
                  pl.BlockSpec((o, 2 * c_real), lambda i: (0, 0))],
        out_specs=[pl.BlockSpec((rblk, o), lambda i: (i, 0)),
                   pl.BlockSpec((1, 1, o), lambda i: (i, 0, 0)),
                   pl.BlockSpec((1, 1, o), lambda i: (i, 0, 0))],
        out_shape=[jax.ShapeDtypeStruct((n, o), jnp.float32),
                   jax.ShapeDtypeStruct((nb, 1, o), jnp.float32),
                   jax.ShapeDtypeStruct((nb, 1, o), jnp.float32)],
    )(fj3, f, w)


def _edge_finalize_body(my_ref, ps_ref, pq_ref, o_ref):
    my = my_ref[...]
    nk = float(NPTS * KNB)
    mean = jnp.sum(ps_ref[...], axis=0) / nk
    ey2 = jnp.sum(pq_ref[...], axis=0) / nk
    var = ey2 - mean * mean
    y = (my - mean) * lax.rsqrt(var + 1e-5)
    o_ref[...] = jnp.where(y >= 0.0, y, 0.2 * y)


def _edgeconv_exact(f, c_real, w, idxt):
    n, cp = f.shape
    fj = _gather_rows_fn(cp)(idxt.reshape(-1), f)
    fj3 = fj.reshape(KNB, n, cp)
    my, ps, pq = _edge_exact_stage1(fj3, f, w, c_real)
    return pl.pallas_call(
        _edge_finalize_body,
        out_shape=jax.ShapeDtypeStruct(my.shape, jnp.float32),
    )(my, ps, pq)


def _combine_body(m_ref, s1_ref, s2_ref, h_ref, o_ref):
    m = m_ref[...]
    s1 = s1_ref[...]
    s2 = s2_ref[...]
    h = h_ref[...]
    nk = float(NPTS * KNB)
    sum_s1 = jnp.sum(s1, axis=0, keepdims=True)
    sum_s2 = jnp.sum(s2, axis=0, keepdims=True)
    sum_h = jnp.sum(h, axis=0, keepdims=True)
    sum_h2 = jnp.sum(h * h, axis=0, keepdims=True)
    sum_hs1 = jnp.sum(h * s1, axis=0, keepdims=True)
    mean = (sum_s1 + KNB * sum_h) / nk
    ey2 = (sum_s2 + 2.0 * sum_hs1 + KNB * sum_h2) / nk
    var = ey2 - mean * mean
    y = (h + m - mean) * lax.rsqrt(var + 1e-5)
    o_ref[...] = jnp.where(y >= 0.0, y, 0.2 * y)


def _combine(m, s1, s2, h):
    return pl.pallas_call(
        _combine_body,
        out_shape=jax.ShapeDtypeStruct(h.shape, jnp.float32),
    )(m, s1, s2, h)


def _edgeconv(f, w1, w2, idxt):
    g, h = _gh(f, w1, w2)
    m, s1, s2 = _gather_stats_fn(w1.shape[0])(idxt.reshape(-1), g)
    return _combine(m, s1, s2, h)


def _conv1d(parts, w):
    n = parts[0].shape[0]
    o = w.shape[0]
    nparts = len(parts)

    def body(*refs):
        w_ref = refs[nparts]
        o_ref = refs[nparts + 1]
        wv = w_ref[...]
        y = None
        c0 = 0
        for i in range(nparts):
            f = refs[i][...]
            c = f.shape[1]
            t = lax.dot_general(f, wv[:, c0:c0 + c], (((1,), (1,)), ((), ())),
                                preferred_element_type=jnp.float32)
            y = t if y is None else y + t
            c0 += c
        mean = jnp.sum(y, axis=0, keepdims=True) / n
        ey2 = jnp.sum(y * y, axis=0, keepdims=True) / n
        var = ey2 - mean * mean
        yv = (y - mean) * lax.rsqrt(var + 1e-5)
        o_ref[...] = jnp.where(yv >= 0.0, yv, 0.2 * yv)

    return pl.pallas_call(
        body,
        out_shape=jax.ShapeDtypeStruct((n, o), jnp.float32),
    )(*parts, w)


def _split_pad(w, c, cp):
    pad = ((0, 0), (0, cp - c))
    return jnp.pad(w[:, :c], pad), jnp.pad(w[:, c:], pad)


def kernel(x, W1s, W2s, W3s, W1c, W2c, W3c, Was, Wac, Wa):
    sem16 = jnp.pad(x[:, :10], ((0, 0), (0, 6)))
    cen16 = jnp.pad(x[:, 13:16], ((0, 0), (0, 13)))
    cen8 = cen16[:, :8]

    idx0t = jnp.transpose(_knn(cen8))

    s1 = _edgeconv_exact(sem16, 10, W1s, idx0t)
    s2 = _edgeconv_exact(s1, 64, W2s, jnp.transpose(_knn(s1)))
    s3 = _edgeconv(s2, W3s[:, :64], W3s[:, 64:], jnp.transpose(_knn(s2)))
    sf = _conv1d([s1, s2, s3], Was)

    c1 = _edgeconv_exact(cen16, 3, W1c, idx0t)
    c2 = _edgeconv_exact(c1, 64, W2c, jnp.transpose(_knn(c1)))
    c3 = _edgeconv(c2, W3c[:, :64], W3c[:, 64:], jnp.transpose(_knn(c2)))
    cf = _conv1d([c1, c2, c3], Wac)

    return _conv1d([sf, cf], Wa)
